# Initial kernel scaffold; baseline (speedup 1.0000x reference)
import jax
import jax.numpy as jnp
from jax.experimental import pallas as pl


def kernel(inputs, neighbor_indices):
    raise NotImplementedError("write your pallas kernel here")


# speedup vs baseline: 1.1336x; 1.1336x over previous
import functools

import jax
import jax.numpy as jnp
from jax import lax
from jax.experimental import pallas as pl
from jax.experimental.pallas import tpu as pltpu
from jax.experimental.pallas import tpu_sc as plsc


def kernel(inputs, neighbor_indices):
    B, L, T, C = inputs.shape
    _, K = neighbor_indices.shape
    D = T * C
    BL = B * L
    R = BL * K

    info = plsc.get_sparse_core_info()
    NC, NS = info.num_cores, info.num_subcores
    NW = NC * NS
    RPW = R // NW
    WPB = NW // B
    CH = 48
    NCH = RPW // CH
    ZROW = BL

    table = jnp.concatenate(
        [inputs.reshape(BL, D), jnp.zeros((8, D), inputs.dtype)], axis=0)
    nidx_flat = neighbor_indices.reshape(L * K)

    mesh = plsc.VectorSubcoreMesh(core_axis_name="c", subcore_axis_name="s")

    @functools.partial(
        pl.kernel,
        mesh=mesh,
        out_type=jax.ShapeDtypeStruct((R, D), inputs.dtype),
        scratch_types=[
            pltpu.VMEM((RPW,), jnp.int32),
            pltpu.VMEM((RPW,), jnp.int32),
            pltpu.VMEM((CH, D), jnp.float32),
            pltpu.VMEM((CH, D), jnp.float32),
            pltpu.SemaphoreType.DMA,
            pltpu.SemaphoreType.DMA,
            pltpu.SemaphoreType.DMA,
            pltpu.SemaphoreType.DMA,
        ],
    )
    def gather_k(table_h, nidx_h, out_h, raw_v, gidx_v, b0, b1,
                 gs0, gs1, ws0, ws1):
        wid = lax.axis_index("s") * NC + lax.axis_index("c")
        b = wid // WPB
        base = wid * RPW
        nbase = (wid % WPB) * RPW
        pltpu.sync_copy(nidx_h.at[pl.ds(nbase, RPW)], raw_v)
        bL = b * L
        for i in range(RPW // 16):
            v = raw_v[pl.ds(i * 16, 16)]
            gidx_v[pl.ds(i * 16, 16)] = jnp.where(v < 0, ZROW, v + bL)

        bufs = (b0, b1)
        gsems = (gs0, gs1)
        wsems = (ws0, ws1)
        gh = [None, None]
        wh = [None, None]
        gh[0] = pltpu.async_copy(
            table_h.at[gidx_v.at[pl.ds(0, CH)]], bufs[0], gsems[0])
        for c in range(NCH):
            j = c & 1
            gh[j].wait()
            wh[j] = pltpu.async_copy(
                bufs[j], out_h.at[pl.ds(base + c * CH, CH)], wsems[j])
            if c + 1 < NCH:
                k2 = 1 - j
                if wh[k2] is not None:
                    wh[k2].wait()
                gh[k2] = pltpu.async_copy(
                    table_h.at[gidx_v.at[pl.ds((c + 1) * CH, CH)]],
                    bufs[k2], gsems[k2])
        for j in range(2):
            if wh[j] is not None:
                wh[j].wait()

    out2d = gather_k(table, nidx_flat)
    return out2d.reshape(B, L, K, T, C)
